# Initial kernel scaffold; baseline (speedup 1.0000x reference)
import jax
import jax.numpy as jnp
from jax.experimental import pallas as pl


def kernel(pts, W1, b1, W2, b2, W3, b3, W4, b4):
    raise NotImplementedError("write your pallas kernel here")


# speedup vs baseline: 1.3948x; 1.3948x over previous
import functools

import jax
import jax.numpy as jnp
from jax.experimental import pallas as pl

N_POINTS = 2048
K = 16
R = 256

_HI = jax.lax.Precision.HIGHEST


def _bf16_dot(a, b):
    return jax.lax.dot(a, b, preferred_element_type=jnp.float32)


def _stage1_kernel(pts_ref, ptsT_ref, w1_ref, b1_ref, w2_ref, b2_ref,
                   idx_ref, rel_ref, f1_ref):
    pts_r = pts_ref[0]
    ptsT = ptsT_ref[0]
    sq_r = jnp.sum(pts_r * pts_r, axis=1, keepdims=True)
    sq_all = jnp.sum(ptsT * ptsT, axis=0, keepdims=True)
    cross = _bf16_dot(pts_r, ptsT)
    d2 = sq_r + sq_all - 2.0 * cross
    lane_iota = jax.lax.broadcasted_iota(jnp.int32, (R, N_POINTS), 1)
    pts_all = ptsT.T
    w1 = w1_ref[...]
    b1 = b1_ref[...]
    w2 = w2_ref[...]
    b2 = b2_ref[...]
    f1 = jnp.full((R, 128), -jnp.inf, jnp.float32)
    for j in range(K):
        m = jnp.min(d2, axis=1, keepdims=True)
        amin = jnp.min(jnp.where(d2 == m, lane_iota, N_POINTS),
                       axis=1, keepdims=True)
        onehot = (lane_iota == amin)
        d2 = jnp.where(onehot, jnp.float32(jnp.inf), d2)
        idx_ref[0, :, j] = amin[:, 0]
        npos = jax.lax.dot(onehot.astype(jnp.float32), pts_all,
                           precision=_HI,
                           preferred_element_type=jnp.float32)
        rel = npos - pts_r
        rel_ref[0, :, j, :] = rel
        h = jnp.maximum(_bf16_dot(rel, w1) + b1, 0.0)
        h = jnp.maximum(_bf16_dot(h, w2) + b2, 0.0)
        f1 = jnp.maximum(f1, h)
    f1_ref[0] = f1


def _stage2_kernel(idx_ref, rel_ref, f1all_ref, w3_ref, b3_ref,
                   w4_ref, b4_ref, fgeo_ref, g_ref):
    t = pl.program_id(1)
    f1_all = f1all_ref[0]
    w3r = w3_ref[0:3, :]
    w3f = w3_ref[3:, :]
    b3 = b3_ref[...]
    w4 = w4_ref[...]
    b4 = b4_ref[...]
    lane_iota = jax.lax.broadcasted_iota(jnp.int32, (R, N_POINTS), 1)
    fgeo = jnp.full((R, 256), -jnp.inf, jnp.float32)
    for j in range(K):
        amin = idx_ref[0, :, j][:, None]
        onehot = (lane_iota == amin).astype(jnp.float32)
        neigh_f = jax.lax.dot(onehot, f1_all, precision=_HI,
                              preferred_element_type=jnp.float32)
        rel = rel_ref[0, :, j, :]
        h = _bf16_dot(rel, w3r) + _bf16_dot(neigh_f, w3f) + b3
        h = jnp.maximum(h, 0.0)
        h = jnp.maximum(_bf16_dot(h, w4) + b4, 0.0)
        fgeo = jnp.maximum(fgeo, h)
    fgeo_ref[0] = fgeo

    @pl.when(t == 0)
    def _init():
        g_ref[...] = jnp.zeros_like(g_ref)

    g_ref[0, 0] += jnp.sum(fgeo, axis=0) / N_POINTS


@jax.jit
def kernel(pts, W1, b1, W2, b2, W3, b3, W4, b4):
    B, N, _ = pts.shape
    ntiles = N // R
    ptsT = jnp.transpose(pts, (0, 2, 1))

    idx, rel, f1 = pl.pallas_call(
        _stage1_kernel,
        grid=(B, ntiles),
        in_specs=[
            pl.BlockSpec((1, R, 3), lambda b, t: (b, t, 0)),
            pl.BlockSpec((1, 3, N), lambda b, t: (b, 0, 0)),
            pl.BlockSpec((3, 64), lambda b, t: (0, 0)),
            pl.BlockSpec((64,), lambda b, t: (0,)),
            pl.BlockSpec((64, 128), lambda b, t: (0, 0)),
            pl.BlockSpec((128,), lambda b, t: (0,)),
        ],
        out_specs=[
            pl.BlockSpec((1, R, K), lambda b, t: (b, t, 0)),
            pl.BlockSpec((1, R, K, 3), lambda b, t: (b, t, 0, 0)),
            pl.BlockSpec((1, R, 128), lambda b, t: (b, t, 0)),
        ],
        out_shape=[
            jax.ShapeDtypeStruct((B, N, K), jnp.int32),
            jax.ShapeDtypeStruct((B, N, K, 3), jnp.float32),
            jax.ShapeDtypeStruct((B, N, 128), jnp.float32),
        ],
    )(pts, ptsT, W1, b1, W2, b2)

    F_geo, g = pl.pallas_call(
        _stage2_kernel,
        grid=(B, ntiles),
        in_specs=[
            pl.BlockSpec((1, R, K), lambda b, t: (b, t, 0)),
            pl.BlockSpec((1, R, K, 3), lambda b, t: (b, t, 0, 0)),
            pl.BlockSpec((1, N, 128), lambda b, t: (b, 0, 0)),
            pl.BlockSpec((3 + 128, 128), lambda b, t: (0, 0)),
            pl.BlockSpec((128,), lambda b, t: (0,)),
            pl.BlockSpec((128, 256), lambda b, t: (0, 0)),
            pl.BlockSpec((256,), lambda b, t: (0,)),
        ],
        out_specs=[
            pl.BlockSpec((1, R, 256), lambda b, t: (b, t, 0)),
            pl.BlockSpec((1, 1, 256), lambda b, t: (b, 0, 0)),
        ],
        out_shape=[
            jax.ShapeDtypeStruct((B, N, 256), jnp.float32),
            jax.ShapeDtypeStruct((B, 1, 256), jnp.float32),
        ],
    )(idx, rel, f1, W3, b3, W4, b4)

    return (F_geo, g[:, 0, :])


# speedup vs baseline: 10.7870x; 7.7335x over previous
import functools

import jax
import jax.numpy as jnp
from jax.experimental import pallas as pl
from jax.experimental.pallas import tpu as pltpu
from jax.experimental.pallas import tpu_sc as plsc

N_POINTS = 2048
K = 16
R = 256
_GW = 128


def _dot(a, b):
    return jax.lax.dot(a, b, preferred_element_type=jnp.float32)


def _knn_kernel(pts_ref, ptsT_ref, gidx_ref):
    b = pl.program_id(0)
    pts_r = pts_ref[0]
    ptsT = ptsT_ref[0]
    sq_r = jnp.sum(pts_r * pts_r, axis=1, keepdims=True)
    sq_all = jnp.sum(ptsT * ptsT, axis=0, keepdims=True)
    d2 = sq_r + sq_all - 2.0 * _dot(pts_r, ptsT)
    lane_iota = jax.lax.broadcasted_iota(jnp.int32, (R, N_POINTS), 1)
    base = b * N_POINTS
    for j in range(K):
        m = jnp.min(d2, axis=1, keepdims=True)
        amin = jnp.min(jnp.where(d2 == m, lane_iota, N_POINTS),
                       axis=1, keepdims=True)
        d2 = jnp.where(lane_iota == amin, jnp.float32(jnp.inf), d2)
        gidx_ref[0, :, j] = amin[:, 0] + base


def _sc_gather(table, gidx_flat, n_rows, n_cols):
    mesh = plsc.VectorSubcoreMesh(core_axis_name="core",
                                  subcore_axis_name="subcore")

    @functools.partial(
        pl.kernel,
        out_type=jax.ShapeDtypeStruct((n_rows, n_cols), table.dtype),
        mesh=mesh,
        compiler_params=pltpu.CompilerParams(use_tc_tiling_on_sc=False),
    )
    def gather_kernel(x_hbm, i_hbm, o_hbm):
        def body(i_vmem, o_vmem):
            pltpu.sync_copy(x_hbm.at[i_vmem.at[0]], o_vmem)

        pltpu.emit_pipeline(
            body,
            grid=(n_rows // _GW,),
            in_specs=[pl.BlockSpec((1, _GW), index_map=lambda i: (0, i))],
            out_specs=[pl.BlockSpec((_GW, n_cols),
                                    index_map=lambda i: (i, 0))],
            core_axis_name=("core", "subcore"),
            dimension_semantics=(pltpu.PARALLEL,),
        )(i_hbm, o_hbm)

    return gather_kernel(table, gidx_flat.reshape(1, n_rows))


def _stage1_kernel(npos_ref, pts_ref, w1_ref, b1_ref, w2_ref, b2_ref,
                   f1_ref):
    npos = npos_ref[0, :, :, 0:3]
    rel = npos - pts_ref[0][:, None, :]
    rel_flat = rel.reshape(R * K, 3)
    h = jnp.maximum(_dot(rel_flat, w1_ref[...]) + b1_ref[...], 0.0)
    h = jnp.maximum(_dot(h, w2_ref[...]) + b2_ref[...], 0.0)
    f1_ref[0] = jnp.max(h.reshape(R, K, 128), axis=1)


def _stage2_kernel(npos_ref, pts_ref, nf_ref, w3_ref, b3_ref,
                   w4_ref, b4_ref, fgeo_ref, g_ref):
    t = pl.program_id(1)
    npos = npos_ref[0, :, :, 0:3]
    rel = npos - pts_ref[0][:, None, :]
    rel_flat = rel.reshape(R * K, 3)
    nf_flat = nf_ref[0].reshape(R * K, 128)
    w3r = w3_ref[0:3, :]
    w3f = w3_ref[3:, :]
    h = _dot(rel_flat, w3r) + _dot(nf_flat, w3f) + b3_ref[...]
    h = jnp.maximum(h, 0.0)
    h = jnp.maximum(_dot(h, w4_ref[...]) + b4_ref[...], 0.0)
    fgeo = jnp.max(h.reshape(R, K, 256), axis=1)
    fgeo_ref[0] = fgeo

    @pl.when(t == 0)
    def _init():
        g_ref[...] = jnp.zeros_like(g_ref)

    g_ref[0, 0] += jnp.sum(fgeo, axis=0) / N_POINTS


@jax.jit
def kernel(pts, W1, b1, W2, b2, W3, b3, W4, b4):
    B, N, _ = pts.shape
    ntiles = N // R
    M = B * N * K
    ptsT = jnp.transpose(pts, (0, 2, 1))

    gidx = pl.pallas_call(
        _knn_kernel,
        grid=(B, ntiles),
        in_specs=[
            pl.BlockSpec((1, R, 3), lambda b, t: (b, t, 0)),
            pl.BlockSpec((1, 3, N), lambda b, t: (b, 0, 0)),
        ],
        out_specs=pl.BlockSpec((1, R, K), lambda b, t: (b, t, 0)),
        out_shape=jax.ShapeDtypeStruct((B, N, K), jnp.int32),
    )(pts, ptsT)

    gidx_flat = gidx.reshape(M)

    pts16 = jnp.pad(pts.reshape(B * N, 3), ((0, 0), (0, 13)))
    npos = _sc_gather(pts16, gidx_flat, M, 16).reshape(B, N, K, 16)

    f1 = pl.pallas_call(
        _stage1_kernel,
        grid=(B, ntiles),
        in_specs=[
            pl.BlockSpec((1, R, K, 16), lambda b, t: (b, t, 0, 0)),
            pl.BlockSpec((1, R, 3), lambda b, t: (b, t, 0)),
            pl.BlockSpec((3, 64), lambda b, t: (0, 0)),
            pl.BlockSpec((64,), lambda b, t: (0,)),
            pl.BlockSpec((64, 128), lambda b, t: (0, 0)),
            pl.BlockSpec((128,), lambda b, t: (0,)),
        ],
        out_specs=pl.BlockSpec((1, R, 128), lambda b, t: (b, t, 0)),
        out_shape=jax.ShapeDtypeStruct((B, N, 128), jnp.float32),
    )(npos, pts, W1, b1, W2, b2)

    nf = _sc_gather(f1.reshape(B * N, 128), gidx_flat, M, 128)
    nf = nf.reshape(B, N, K, 128)

    F_geo, g = pl.pallas_call(
        _stage2_kernel,
        grid=(B, ntiles),
        in_specs=[
            pl.BlockSpec((1, R, K, 16), lambda b, t: (b, t, 0, 0)),
            pl.BlockSpec((1, R, 3), lambda b, t: (b, t, 0)),
            pl.BlockSpec((1, R, K, 128), lambda b, t: (b, t, 0, 0)),
            pl.BlockSpec((3 + 128, 128), lambda b, t: (0, 0)),
            pl.BlockSpec((128,), lambda b, t: (0,)),
            pl.BlockSpec((128, 256), lambda b, t: (0, 0)),
            pl.BlockSpec((256,), lambda b, t: (0,)),
        ],
        out_specs=[
            pl.BlockSpec((1, R, 256), lambda b, t: (b, t, 0)),
            pl.BlockSpec((1, 1, 256), lambda b, t: (b, 0, 0)),
        ],
        out_shape=[
            jax.ShapeDtypeStruct((B, N, 256), jnp.float32),
            jax.ShapeDtypeStruct((B, 1, 256), jnp.float32),
        ],
    )(npos, pts, nf, W3, b3, W4, b4)

    return (F_geo, g[:, 0, :])


# speedup vs baseline: 11.2134x; 1.0395x over previous
import functools

import jax
import jax.numpy as jnp
from jax.experimental import pallas as pl
from jax.experimental.pallas import tpu as pltpu
from jax.experimental.pallas import tpu_sc as plsc

N_POINTS = 2048
K = 16
R = 256
_GW = 128


def _dot(a, b):
    return jax.lax.dot(a, b, preferred_element_type=jnp.float32)


def _knn_kernel(pts_ref, ptsT_ref, gidx_ref):
    b = pl.program_id(0)
    pts_r = pts_ref[0]
    ptsT = ptsT_ref[0]
    sq_r = jnp.sum(pts_r * pts_r, axis=1, keepdims=True)
    sq_all = jnp.sum(ptsT * ptsT, axis=0, keepdims=True)
    d2 = sq_r + sq_all - 2.0 * _dot(pts_r, ptsT)
    lane_iota = jax.lax.broadcasted_iota(jnp.int32, (R, N_POINTS), 1)
    base = b * N_POINTS
    for j in range(K):
        m = jnp.min(d2, axis=1, keepdims=True)
        amin = jnp.min(jnp.where(d2 == m, lane_iota, N_POINTS),
                       axis=1, keepdims=True)
        d2 = jnp.where(lane_iota == amin, jnp.float32(jnp.inf), d2)
        gidx_ref[0, :, j] = amin[:, 0] + base


def _sc_gather(table, gidx_flat, n_rows, n_cols):
    mesh = plsc.VectorSubcoreMesh(core_axis_name="core",
                                  subcore_axis_name="subcore")

    @functools.partial(
        pl.kernel,
        out_type=jax.ShapeDtypeStruct((n_rows, n_cols), table.dtype),
        mesh=mesh,
        compiler_params=pltpu.CompilerParams(use_tc_tiling_on_sc=False),
    )
    def gather_kernel(x_hbm, i_hbm, o_hbm):
        def body(i_vmem, o_vmem):
            pltpu.sync_copy(x_hbm.at[i_vmem.at[0]], o_vmem)

        pltpu.emit_pipeline(
            body,
            grid=(n_rows // _GW,),
            in_specs=[pl.BlockSpec((1, _GW), index_map=lambda i: (0, i))],
            out_specs=[pl.BlockSpec((_GW, n_cols),
                                    index_map=lambda i: (i, 0))],
            core_axis_name=("core", "subcore"),
            dimension_semantics=(pltpu.PARALLEL,),
        )(i_hbm, o_hbm)

    return gather_kernel(table, gidx_flat.reshape(1, n_rows))


def _stage1_kernel(npos_ref, pts_ref, w1_ref, b1_ref, w2_ref, b2_ref,
                   f1_ref):
    npos = npos_ref[0, :, :, 0:3]
    rel = npos - pts_ref[0][:, None, :]
    rel_flat = rel.reshape(R * K, 3)
    h = jnp.maximum(_dot(rel_flat, w1_ref[...]) + b1_ref[...], 0.0)
    h = jnp.maximum(_dot(h, w2_ref[...]) + b2_ref[...], 0.0)
    f1_ref[0] = jnp.max(h.reshape(R, K, 128), axis=1)


def _stage2_kernel(npos_ref, pts_ref, nf_ref, w3_ref, b3_ref,
                   w4_ref, b4_ref, fgeo_ref, g_ref):
    t = pl.program_id(1)
    npos = npos_ref[0, :, :, 0:3]
    rel = npos - pts_ref[0][:, None, :]
    rel_flat = rel.reshape(R * K, 3)
    nf_flat = nf_ref[0].reshape(R * K, 128)
    w3r = w3_ref[0:3, :]
    w3f = w3_ref[3:, :]
    h = _dot(rel_flat, w3r) + _dot(nf_flat, w3f) + b3_ref[...]
    h = jnp.maximum(h, 0.0)
    h = jnp.maximum(_dot(h, w4_ref[...]) + b4_ref[...], 0.0)
    fgeo = jnp.max(h.reshape(R, K, 256), axis=1)
    fgeo_ref[0] = fgeo

    @pl.when(t == 0)
    def _init():
        g_ref[...] = jnp.zeros_like(g_ref)

    g_ref[0, 0] += jnp.sum(fgeo, axis=0) / N_POINTS


def _encode_one(pts, W1, b1, W2, b2, W3, b3, W4, b4):
    B, N, _ = pts.shape
    ntiles = N // R
    M = B * N * K
    ptsT = jnp.transpose(pts, (0, 2, 1))

    gidx = pl.pallas_call(
        _knn_kernel,
        grid=(B, ntiles),
        in_specs=[
            pl.BlockSpec((1, R, 3), lambda b, t: (b, t, 0)),
            pl.BlockSpec((1, 3, N), lambda b, t: (b, 0, 0)),
        ],
        out_specs=pl.BlockSpec((1, R, K), lambda b, t: (b, t, 0)),
        out_shape=jax.ShapeDtypeStruct((B, N, K), jnp.int32),
    )(pts, ptsT)

    gidx_flat = gidx.reshape(M)

    pts16 = jnp.pad(pts.reshape(B * N, 3), ((0, 0), (0, 13)))
    npos = _sc_gather(pts16, gidx_flat, M, 16).reshape(B, N, K, 16)

    f1 = pl.pallas_call(
        _stage1_kernel,
        grid=(B, ntiles),
        in_specs=[
            pl.BlockSpec((1, R, K, 16), lambda b, t: (b, t, 0, 0)),
            pl.BlockSpec((1, R, 3), lambda b, t: (b, t, 0)),
            pl.BlockSpec((3, 64), lambda b, t: (0, 0)),
            pl.BlockSpec((64,), lambda b, t: (0,)),
            pl.BlockSpec((64, 128), lambda b, t: (0, 0)),
            pl.BlockSpec((128,), lambda b, t: (0,)),
        ],
        out_specs=pl.BlockSpec((1, R, 128), lambda b, t: (b, t, 0)),
        out_shape=jax.ShapeDtypeStruct((B, N, 128), jnp.float32),
    )(npos, pts, W1, b1, W2, b2)

    nf = _sc_gather(f1.reshape(B * N, 128), gidx_flat, M, 128)
    nf = nf.reshape(B, N, K, 128)

    F_geo, g = pl.pallas_call(
        _stage2_kernel,
        grid=(B, ntiles),
        in_specs=[
            pl.BlockSpec((1, R, K, 16), lambda b, t: (b, t, 0, 0)),
            pl.BlockSpec((1, R, 3), lambda b, t: (b, t, 0)),
            pl.BlockSpec((1, R, K, 128), lambda b, t: (b, t, 0, 0)),
            pl.BlockSpec((3 + 128, 128), lambda b, t: (0, 0)),
            pl.BlockSpec((128,), lambda b, t: (0,)),
            pl.BlockSpec((128, 256), lambda b, t: (0, 0)),
            pl.BlockSpec((256,), lambda b, t: (0,)),
        ],
        out_specs=[
            pl.BlockSpec((1, R, 256), lambda b, t: (b, t, 0)),
            pl.BlockSpec((1, 1, 256), lambda b, t: (b, 0, 0)),
        ],
        out_shape=[
            jax.ShapeDtypeStruct((B, N, 256), jnp.float32),
            jax.ShapeDtypeStruct((B, 1, 256), jnp.float32),
        ],
    )(npos, pts, nf, W3, b3, W4, b4)

    return (F_geo, g[:, 0, :])


@jax.jit
def kernel(pts, W1, b1, W2, b2, W3, b3, W4, b4):
    outs = [_encode_one(pts[b:b + 1], W1, b1, W2, b2, W3, b3, W4, b4)
            for b in range(pts.shape[0])]
    F_geo = jnp.concatenate([o[0] for o in outs], axis=0)
    g = jnp.concatenate([o[1] for o in outs], axis=0)
    return (F_geo, g)
